# baseline (speedup 1.0000x reference)
import functools

import jax
import jax.numpy as jnp
from jax.experimental import pallas as pl
from jax.experimental.pallas import tpu as pltpu
from jax.experimental.pallas import tpu_sc as plsc

_KT = 1024


def _qa_body(nk, kt_last, z_ref, pw_ref, pb_ref, emb_ref,
             ft_out, f2_out, lse_out, ind_out, diff_out,
             ft_s, f2_s, m_s, s_s, bv_s, bi_s, acc_s):
    b = pl.program_id(0)
    kt = pl.program_id(1)
    nb = pl.num_programs(0)
    hw = ft_s.shape[1]

    @pl.when(kt == 0)
    def _init():
        ft = jnp.dot(pw_ref[...], z_ref[0],
                     preferred_element_type=jnp.float32) + pb_ref[...]
        ft_s[...] = ft
        ft_out[...] = ft
        f2 = jnp.sum(ft * ft, axis=0, keepdims=True)
        f2_s[...] = f2
        f2_out[0] = f2
        m_s[...] = jnp.full((1, hw), -jnp.inf, jnp.float32)
        s_s[...] = jnp.zeros((1, hw), jnp.float32)
        bv_s[...] = jnp.full((1, hw), -jnp.inf, jnp.float32)
        bi_s[...] = jnp.zeros((1, hw), jnp.int32)

    e = emb_ref[...]
    e2 = jnp.sum(e * e, axis=1, keepdims=True)
    mm = jnp.dot(e, ft_s[...], preferred_element_type=jnp.float32)
    v = -((f2_s[...] - 2.0 * mm) + e2)

    tmax = jnp.max(v, axis=0, keepdims=True)
    rows = jax.lax.broadcasted_iota(jnp.int32, v.shape, 0)
    big = jnp.int32(nk * _KT)
    idx = jnp.min(jnp.where(v == tmax, rows, big), axis=0, keepdims=True)
    idx = idx + kt * _KT

    m_old = m_s[...]
    m_new = jnp.maximum(m_old, tmax)
    s_s[...] = (s_s[...] * jnp.exp(m_old - m_new)
                + jnp.sum(jnp.exp(v - m_new), axis=0, keepdims=True))
    m_s[...] = m_new

    flip = tmax > bv_s[...]
    bi_s[...] = jnp.where(flip, idx, bi_s[...])
    bv_s[...] = jnp.maximum(bv_s[...], tmax)

    @pl.when(kt == kt_last)
    def _fin():
        lse_out[0] = m_s[...] + jnp.log(s_s[...])
        ind_out[0] = bi_s[...]
        part = -jnp.sum(bv_s[...]).reshape(1, 1)
        tot = jnp.where(b == 0, part, acc_s[...] + part)
        acc_s[...] = tot

        @pl.when(b == nb - 1)
        def _done():
            n_total = nb * hw * ft_s.shape[0]
            diff_out[...] = tot * jnp.float32(12.5 / n_total)


def _qb_body(emb_ref, ft_ref, f2_ref, lse_ref, lp_out):
    e = emb_ref[...]
    e2 = jnp.sum(e * e, axis=1, keepdims=True)
    mm = jnp.dot(e.astype(jnp.bfloat16), ft_ref[...].astype(jnp.bfloat16),
                 preferred_element_type=jnp.float32)
    dist = (f2_ref[0] - 2.0 * mm) + e2
    lp_out[0] = (-dist) - lse_ref[0]


def _qd_body(zq_ref, zqt_out):
    zqt_out[0] = zq_ref[...].T


def _gather_rows(ind2, embed_w):
    nrow, lanes = ind2.shape
    n = nrow * lanes
    k, d = embed_w.shape
    nw = 32
    bpw = n // nw
    chunks = bpw // lanes
    mesh = plsc.VectorSubcoreMesh(core_axis_name="c", subcore_axis_name="s")

    @functools.partial(
        pl.kernel,
        out_type=jax.ShapeDtypeStruct((n, d), jnp.float32),
        mesh=mesh,
        scratch_types=[
            pltpu.VMEM((chunks, lanes), jnp.int32),
            pltpu.VMEM((bpw, d), jnp.float32),
            pltpu.SemaphoreType.DMA,
        ],
    )
    def gk(idx_hbm, tab_hbm, out_hbm, idx_v, rows_v, sem):
        wid = jax.lax.axis_index("s") * 2 + jax.lax.axis_index("c")
        pltpu.sync_copy(idx_hbm.at[pl.ds(wid * chunks, chunks)], idx_v)
        cps = [
            pltpu.async_copy(tab_hbm.at[idx_v.at[j]],
                             rows_v.at[pl.ds(j * lanes, lanes)], sem)
            for j in range(chunks)
        ]
        for cp in cps:
            cp.wait()
        pltpu.sync_copy(rows_v, out_hbm.at[pl.ds(wid * bpw, bpw)])

    return gk(ind2, embed_w)


def kernel(z, proj_w, proj_b, embed_w):
    bz, c, h, w = z.shape
    d = proj_w.shape[0]
    k = embed_w.shape[0]
    hw = h * w
    n = bz * hw
    nk = k // _KT

    z3 = z.reshape(bz, c, hw)
    pb = proj_b.reshape(d, 1)

    ft, f2o, lseo, indo, diffo = pl.pallas_call(
        functools.partial(_qa_body, nk, nk - 1),
        grid=(bz, nk),
        in_specs=[
            pl.BlockSpec((1, c, hw), lambda b, t: (b, 0, 0)),
            pl.BlockSpec((d, c), lambda b, t: (0, 0)),
            pl.BlockSpec((d, 1), lambda b, t: (0, 0)),
            pl.BlockSpec((_KT, d), lambda b, t: (t, 0)),
        ],
        out_specs=[
            pl.BlockSpec((d, hw), lambda b, t: (0, b)),
            pl.BlockSpec((1, 1, hw), lambda b, t: (b, 0, 0)),
            pl.BlockSpec((1, 1, hw), lambda b, t: (b, 0, 0)),
            pl.BlockSpec((1, 1, hw), lambda b, t: (b, 0, 0)),
            pl.BlockSpec((1, 1), lambda b, t: (0, 0)),
        ],
        out_shape=[
            jax.ShapeDtypeStruct((d, n), jnp.float32),
            jax.ShapeDtypeStruct((bz, 1, hw), jnp.float32),
            jax.ShapeDtypeStruct((bz, 1, hw), jnp.float32),
            jax.ShapeDtypeStruct((bz, 1, hw), jnp.int32),
            jax.ShapeDtypeStruct((1, 1), jnp.float32),
        ],
        scratch_shapes=[
            pltpu.VMEM((d, hw), jnp.float32),
            pltpu.VMEM((1, hw), jnp.float32),
            pltpu.VMEM((1, hw), jnp.float32),
            pltpu.VMEM((1, hw), jnp.float32),
            pltpu.VMEM((1, hw), jnp.float32),
            pltpu.VMEM((1, hw), jnp.int32),
            pltpu.VMEM((1, 1), jnp.float32),
        ],
    )(z3, proj_w, pb, embed_w)

    zq_flat = _gather_rows(indo.reshape(n // 128, 128), embed_w)

    lp = pl.pallas_call(
        _qb_body,
        grid=(bz, nk),
        in_specs=[
            pl.BlockSpec((_KT, d), lambda b, t: (t, 0)),
            pl.BlockSpec((d, hw), lambda b, t: (0, b)),
            pl.BlockSpec((1, 1, hw), lambda b, t: (b, 0, 0)),
            pl.BlockSpec((1, 1, hw), lambda b, t: (b, 0, 0)),
        ],
        out_specs=pl.BlockSpec((1, _KT, hw), lambda b, t: (b, t, 0)),
        out_shape=jax.ShapeDtypeStruct((bz, k, hw), jnp.float32),
    )(embed_w, ft, f2o, lseo)

    zqt = pl.pallas_call(
        _qd_body,
        grid=(bz,),
        in_specs=[pl.BlockSpec((hw, d), lambda b: (b, 0))],
        out_specs=pl.BlockSpec((1, d, hw), lambda b: (b, 0, 0)),
        out_shape=jax.ShapeDtypeStruct((bz, d, hw), jnp.float32),
    )(zq_flat)

    z_q = zqt.reshape(bz, d, h, w)
    diff = diffo.reshape(())
    ind = indo.reshape(bz, h, w)
    log_priors = lp.reshape(bz, k, h, w)
    return (z_q, diff, ind, log_priors)


# speedup vs baseline: 1.6398x; 1.6398x over previous
import functools

import jax
import jax.numpy as jnp
from jax.experimental import pallas as pl
from jax.experimental.pallas import tpu as pltpu
from jax.experimental.pallas import tpu_sc as plsc

_KT = 1024
_NT = 1024


def _e2_body(e_ref, e2_out):
    e = e_ref[...]
    e2_out[...] = jnp.sum(e * e, axis=1, keepdims=True)


def _qa_body(kt_last, z_ref, pwt_ref, pb_ref, embt_ref, e2_ref,
             f_out, f2_out, lse_out, ind_out, diff_out,
             f2x_s, f2_s, s_s, bm_s, bi_s, acc_s):
    nb = pl.program_id(0)
    kt = pl.program_id(1)
    nbt = pl.num_programs(0)
    nt = f2x_s.shape[0]

    @pl.when(kt == 0)
    def _init():
        f = jnp.dot(z_ref[...], pwt_ref[...],
                    preferred_element_type=jnp.float32) + pb_ref[...]
        f2x = f + f
        f2x_s[...] = f2x
        f_out[...] = f2x.astype(jnp.bfloat16)
        f2 = jnp.sum(f * f, axis=1, keepdims=True)
        f2_s[...] = f2
        f2_out[...] = f2
        s_s[...] = jnp.zeros((nt, 1), jnp.float32)
        bm_s[...] = jnp.full((nt, 1), jnp.inf, jnp.float32)
        bi_s[...] = jnp.zeros((nt, 1), jnp.int32)

    e = embt_ref[:, pl.ds(kt * _KT, _KT)]
    m2 = jnp.dot(f2x_s[...], e, preferred_element_type=jnp.float32)
    dist = (f2_s[...] - m2) + e2_ref[...]

    tmin = jnp.min(dist, axis=1, keepdims=True)
    cols = jax.lax.broadcasted_iota(jnp.int32, dist.shape, 1)
    big = jnp.int32(2147480000)
    idx = jnp.min(jnp.where(dist == tmin, cols, big), axis=1, keepdims=True)
    idx = idx + kt * _KT

    bm_old = bm_s[...]
    bm_new = jnp.minimum(bm_old, tmin)
    s_s[...] = (s_s[...] * jnp.exp(bm_new - bm_old)
                + jnp.sum(jnp.exp(bm_new - dist), axis=1, keepdims=True))
    bm_s[...] = bm_new
    flip = tmin < bm_old
    bi_s[...] = jnp.where(flip, idx, bi_s[...])

    @pl.when(kt == kt_last)
    def _fin():
        lse_out[...] = jnp.log(s_s[...]) - bm_s[...]
        ind_out[...] = bi_s[...]
        part = jnp.sum(bm_s[...]).reshape(1, 1)
        tot = jnp.where(nb == 0, part, acc_s[...] + part)
        acc_s[...] = tot

        @pl.when(nb == nbt - 1)
        def _done():
            n_total = nbt * nt * f2x_s.shape[1]
            diff_out[...] = tot * jnp.float32(12.5 / n_total)


def _qb_body(embt_ref, f_ref, f2_ref, lse_ref, e2_ref, lp_out, ebf_s):
    nb = pl.program_id(0)
    kt = pl.program_id(1)

    @pl.when(nb == 0)
    def _stage():
        ebf_s[:, pl.ds(kt * _KT, _KT)] = (
            embt_ref[:, pl.ds(kt * _KT, _KT)].astype(jnp.bfloat16))

    e = ebf_s[:, pl.ds(kt * _KT, _KT)]
    m2 = jnp.dot(f_ref[...], e, preferred_element_type=jnp.float32)
    lp_out[...] = ((m2 - f2_ref[...]) - e2_ref[...]) - lse_ref[...]


def _gather_rows(ind2, embed_w):
    nrow, lanes = ind2.shape
    n = nrow * lanes
    k, d = embed_w.shape
    nw = 32
    bpw = n // nw
    chunks = bpw // lanes
    mesh = plsc.VectorSubcoreMesh(core_axis_name="c", subcore_axis_name="s")

    @functools.partial(
        pl.kernel,
        out_type=jax.ShapeDtypeStruct((n, d), jnp.float32),
        mesh=mesh,
        scratch_types=[
            pltpu.VMEM((chunks, lanes), jnp.int32),
            pltpu.VMEM((bpw, d), jnp.float32),
            pltpu.SemaphoreType.DMA,
        ],
    )
    def gk(idx_hbm, tab_hbm, out_hbm, idx_v, rows_v, sem):
        wid = jax.lax.axis_index("s") * 2 + jax.lax.axis_index("c")
        pltpu.sync_copy(idx_hbm.at[pl.ds(wid * chunks, chunks)], idx_v)
        cps = [
            pltpu.async_copy(tab_hbm.at[idx_v.at[j]],
                             rows_v.at[pl.ds(j * lanes, lanes)], sem)
            for j in range(chunks)
        ]
        for cp in cps:
            cp.wait()
        pltpu.sync_copy(rows_v, out_hbm.at[pl.ds(wid * bpw, bpw)])

    return gk(ind2, embed_w)


def kernel(z, proj_w, proj_b, embed_w):
    bz, c, h, w = z.shape
    d = proj_w.shape[0]
    k = embed_w.shape[0]
    n = bz * h * w
    nk = k // _KT
    nb = n // _NT

    z_rows = z.transpose(0, 2, 3, 1).reshape(n, c)
    pwt = proj_w.T
    pb_row = proj_b.reshape(1, d)
    embt = embed_w.T

    e2col = pl.pallas_call(
        _e2_body,
        grid=(nk,),
        in_specs=[pl.BlockSpec((_KT, d), lambda t: (t, 0))],
        out_specs=pl.BlockSpec((_KT, 1), lambda t: (t, 0)),
        out_shape=jax.ShapeDtypeStruct((k, 1), jnp.float32),
    )(embed_w)
    e2row = e2col.reshape(1, k)

    f2x, f2col, lsecol, indcol, diffo = pl.pallas_call(
        functools.partial(_qa_body, nk - 1),
        grid=(nb, nk),
        in_specs=[
            pl.BlockSpec((_NT, c), lambda b, t: (b, 0)),
            pl.BlockSpec((c, d), lambda b, t: (0, 0)),
            pl.BlockSpec((1, d), lambda b, t: (0, 0)),
            pl.BlockSpec((d, k), lambda b, t: (0, 0)),
            pl.BlockSpec((1, _KT), lambda b, t: (0, t)),
        ],
        out_specs=[
            pl.BlockSpec((_NT, d), lambda b, t: (b, 0)),
            pl.BlockSpec((_NT, 1), lambda b, t: (b, 0)),
            pl.BlockSpec((_NT, 1), lambda b, t: (b, 0)),
            pl.BlockSpec((_NT, 1), lambda b, t: (b, 0)),
            pl.BlockSpec((1, 1), lambda b, t: (0, 0)),
        ],
        out_shape=[
            jax.ShapeDtypeStruct((n, d), jnp.bfloat16),
            jax.ShapeDtypeStruct((n, 1), jnp.float32),
            jax.ShapeDtypeStruct((n, 1), jnp.float32),
            jax.ShapeDtypeStruct((n, 1), jnp.int32),
            jax.ShapeDtypeStruct((1, 1), jnp.float32),
        ],
        scratch_shapes=[
            pltpu.VMEM((_NT, d), jnp.float32),
            pltpu.VMEM((_NT, 1), jnp.float32),
            pltpu.VMEM((_NT, 1), jnp.float32),
            pltpu.VMEM((_NT, 1), jnp.float32),
            pltpu.VMEM((_NT, 1), jnp.int32),
            pltpu.VMEM((1, 1), jnp.float32),
        ],
    )(z_rows, pwt, pb_row, embt, e2row)

    ind_flat = indcol.reshape(n)
    zq_flat = _gather_rows(ind_flat.reshape(n // 128, 128), embed_w)

    lp = pl.pallas_call(
        _qb_body,
        grid=(nb, nk),
        in_specs=[
            pl.BlockSpec((d, k), lambda b, t: (0, 0)),
            pl.BlockSpec((_NT, d), lambda b, t: (b, 0)),
            pl.BlockSpec((_NT, 1), lambda b, t: (b, 0)),
            pl.BlockSpec((_NT, 1), lambda b, t: (b, 0)),
            pl.BlockSpec((1, _KT), lambda b, t: (0, t)),
        ],
        out_specs=pl.BlockSpec((_NT, _KT), lambda b, t: (b, t)),
        out_shape=jax.ShapeDtypeStruct((n, k), jnp.float32),
        scratch_shapes=[pltpu.VMEM((d, k), jnp.bfloat16)],
    )(embt, f2x, f2col, lsecol, e2row)

    z_q = zq_flat.reshape(bz, h, w, d).transpose(0, 3, 1, 2)
    log_priors = lp.reshape(bz, h, w, k).transpose(0, 3, 1, 2)
    ind = ind_flat.reshape(bz, h, w)
    diff = diffo.reshape(())
    return (z_q, diff, ind, log_priors)
